# baseline (speedup 1.0000x reference)
import functools

import jax
import jax.numpy as jnp
from jax import lax
from jax.experimental import pallas as pl
from jax.experimental.pallas import tpu as pltpu
from jax.experimental.pallas import tpu_sc as plsc

SCALE = 511.0
B, C, H, W = 2, 128, 512, 512
HW = H * W
N = 131072

NC, NS, L = 2, 16, 16
NW = NC * NS
PTS_PER_W = (B * N) // NW
P = 128
CHUNKS = PTS_PER_W // P

HCHUNK = 4096


def _tr_in_body(g_ref, t_ref):
    t_ref[...] = g_ref[0].T


def _build_table(grid3):
    nblk = HW // HCHUNK
    return pl.pallas_call(
        _tr_in_body,
        grid=(B, nblk),
        in_specs=[pl.BlockSpec((1, C, HCHUNK), lambda b, j: (b, 0, j))],
        out_specs=pl.BlockSpec((HCHUNK, C), lambda b, j: (b * nblk + j, 0)),
        out_shape=jax.ShapeDtypeStruct((B * HW, C), jnp.float32),
    )(grid3)


@functools.partial(
    pl.kernel,
    out_type=jax.ShapeDtypeStruct((B, C, N), jnp.float32),
    mesh=plsc.VectorSubcoreMesh(core_axis_name="c", subcore_axis_name="s"),
    compiler_params=pltpu.CompilerParams(needs_layout_passes=False),
    scratch_types=[
        pltpu.VMEM((2 * P,), jnp.float32),
        pltpu.VMEM((P,), jnp.int32),
        pltpu.VMEM((P,), jnp.int32),
        pltpu.VMEM((P,), jnp.int32),
        pltpu.VMEM((P,), jnp.int32),
        pltpu.VMEM((P,), jnp.float32),
        pltpu.VMEM((P,), jnp.float32),
        pltpu.VMEM((P,), jnp.float32),
        pltpu.VMEM((P,), jnp.float32),
        pltpu.VMEM((P, C), jnp.float32),
        pltpu.VMEM((P, C), jnp.float32),
        pltpu.VMEM((P, C), jnp.float32),
        pltpu.VMEM((P, C), jnp.float32),
        pltpu.VMEM((C, P), jnp.float32),
        pltpu.SemaphoreType.DMA,
    ],
)
def _sc_gather(table, pc_hbm, out, cv, i00, i01, i10, i11,
               w00, w01, w10, w11, r00, r01, r10, r11, oc, sem):
    cid = lax.axis_index("c")
    sid = lax.axis_index("s")
    wid = sid * NC + cid
    b = wid // NS
    lane = wid % NS
    base = lane * PTS_PER_W
    iota = lax.iota(jnp.int32, L)
    boff = b * HW

    def chunk(g, carry):
        n0 = base + g * P
        pltpu.sync_copy(pc_hbm.at[b, pl.ds(2 * n0, 2 * P)], cv)
        for t in range(P // L):
            sl = pl.ds(t * L, L)
            hv = plsc.load_gather(cv, [t * (2 * L) + iota * 2]) * SCALE
            wv = plsc.load_gather(cv, [t * (2 * L) + iota * 2 + 1]) * SCALE
            h0i = hv.astype(jnp.int32)
            w0i = wv.astype(jnp.int32)
            wh1 = hv - h0i.astype(jnp.float32)
            ww1 = wv - w0i.astype(jnp.float32)
            wh0 = 1.0 - wh1
            ww0 = 1.0 - ww1
            r0 = boff + h0i * W + w0i
            i00[sl] = r0
            i01[sl] = r0 + 1
            i10[sl] = r0 + W
            i11[sl] = r0 + (W + 1)
            w00[sl] = wh0 * ww0
            w01[sl] = wh0 * ww1
            w10[sl] = wh1 * ww0
            w11[sl] = wh1 * ww1
        d1 = pltpu.async_copy(table.at[i00], r00, sem)
        d2 = pltpu.async_copy(table.at[i01], r01, sem)
        d3 = pltpu.async_copy(table.at[i10], r10, sem)
        d4 = pltpu.async_copy(table.at[i11], r11, sem)
        d1.wait()
        d2.wait()
        d3.wait()
        d4.wait()

        for t in range(P // L):
            sl = pl.ds(t * L, L)
            a00 = w00[sl]
            a01 = w01[sl]
            a10 = w10[sl]
            a11 = w11[sl]
            pvec = t * L + iota

            def ch(c, carry2):
                cvec = jnp.full((L,), c, jnp.int32)
                v00 = plsc.load_gather(r00, [pvec, cvec])
                v01 = plsc.load_gather(r01, [pvec, cvec])
                v10 = plsc.load_gather(r10, [pvec, cvec])
                v11 = plsc.load_gather(r11, [pvec, cvec])
                oc[c, sl] = v00 * a00 + v01 * a01 + v10 * a10 + v11 * a11
                return carry2

            lax.fori_loop(0, C, ch, 0, unroll=2)

        pltpu.sync_copy(oc, out.at[b, :, pl.ds(n0, P)])
        return carry

    lax.fori_loop(0, CHUNKS, chunk, 0)


def kernel(grid_in, pcds_ind):
    grid3 = grid_in.reshape(B, C, HW)
    table = _build_table(grid3)
    pc = pcds_ind.reshape(B, 2 * N)
    out = _sc_gather(table, pc)
    return out[..., None]


# speedup vs baseline: 1.7380x; 1.7380x over previous
import functools

import jax
import jax.numpy as jnp
from jax import lax
from jax.experimental import pallas as pl
from jax.experimental.pallas import tpu as pltpu
from jax.experimental.pallas import tpu_sc as plsc

SCALE = 511.0
B, C, H, W = 2, 128, 512, 512
HW = H * W
N = 131072

NC, NS, L = 2, 16, 16
NW = NC * NS
PTS_PER_W = (B * N) // NW
P = 64
CHUNKS = PTS_PER_W // P
QUADS = CHUNKS // 4

HCHUNK = 4096


def _tr_in_body(g_ref, t_ref):
    t_ref[...] = g_ref[0].T


def _build_table(grid3):
    nblk = HW // HCHUNK
    return pl.pallas_call(
        _tr_in_body,
        grid=(B, nblk),
        in_specs=[pl.BlockSpec((1, C, HCHUNK), lambda b, j: (b, 0, j))],
        out_specs=pl.BlockSpec((HCHUNK, C), lambda b, j: (b * nblk + j, 0)),
        out_shape=jax.ShapeDtypeStruct((B * HW, C), jnp.float32),
    )(grid3)


def _mk_scratch():
    sets = []
    for _ in range(2):
        sets += [
            pltpu.VMEM((2 * P,), jnp.float32),
            pltpu.VMEM((P,), jnp.int32),
            pltpu.VMEM((P,), jnp.int32),
            pltpu.VMEM((P,), jnp.int32),
            pltpu.VMEM((P,), jnp.int32),
            pltpu.VMEM((P,), jnp.float32),
            pltpu.VMEM((P,), jnp.float32),
            pltpu.VMEM((P, C), jnp.float32),
            pltpu.VMEM((P, C), jnp.float32),
            pltpu.VMEM((P, C), jnp.float32),
            pltpu.VMEM((P, C), jnp.float32),
            pltpu.SemaphoreType.DMA,
        ]
    for _ in range(2):
        sets += [
            pltpu.VMEM((C, 2 * P), jnp.float32),
            pltpu.SemaphoreType.DMA,
        ]
    return sets


@functools.partial(
    pl.kernel,
    out_type=jax.ShapeDtypeStruct((B, C, N), jnp.float32),
    mesh=plsc.VectorSubcoreMesh(core_axis_name="c", subcore_axis_name="s"),
    compiler_params=pltpu.CompilerParams(needs_layout_passes=False),
    scratch_types=_mk_scratch(),
)
def _sc_gather(table, pc_hbm, out, *scr):
    cid = lax.axis_index("c")
    sid = lax.axis_index("s")
    wid = sid * NC + cid
    b = wid // NS
    lane = wid % NS
    base = lane * PTS_PER_W
    iota = lax.iota(jnp.int32, L)
    boff = b * HW
    sets = [scr[0:12], scr[12:24]]
    oc0, osem0, oc1, osem1 = scr[24:28]
    ocs = [(oc0, osem0), (oc1, osem1)]

    def fire(g, s):
        cv, i00, i01, i10, i11, wh, ww, r00, r01, r10, r11, gsem = s
        n0 = base + g * P
        pltpu.sync_copy(pc_hbm.at[b, pl.ds(2 * n0, 2 * P)], cv)
        for t in range(P // L):
            sl = pl.ds(t * L, L)
            hv = plsc.load_gather(cv, [t * (2 * L) + iota * 2]) * SCALE
            wv = plsc.load_gather(cv, [t * (2 * L) + iota * 2 + 1]) * SCALE
            h0i = hv.astype(jnp.int32)
            w0i = wv.astype(jnp.int32)
            wh[sl] = hv - h0i.astype(jnp.float32)
            ww[sl] = wv - w0i.astype(jnp.float32)
            r0 = boff + h0i * W + w0i
            i00[sl] = r0
            i01[sl] = r0 + 1
            i10[sl] = r0 + W
            i11[sl] = r0 + (W + 1)
        pltpu.async_copy(table.at[i00], r00, gsem)
        pltpu.async_copy(table.at[i01], r01, gsem)
        pltpu.async_copy(table.at[i10], r10, gsem)
        pltpu.async_copy(table.at[i11], r11, gsem)

    def blend(g, s, oc, poff):
        cv, i00, i01, i10, i11, wh, ww, r00, r01, r10, r11, gsem = s
        pltpu.make_async_copy(table.at[i00], r00, gsem).wait()
        pltpu.make_async_copy(table.at[i01], r01, gsem).wait()
        pltpu.make_async_copy(table.at[i10], r10, gsem).wait()
        pltpu.make_async_copy(table.at[i11], r11, gsem).wait()

        def pt(i, carry):
            iv = jnp.full((L,), i, jnp.int32)
            ah = plsc.load_gather(wh, [iv])
            aw = plsc.load_gather(ww, [iv])
            col = jnp.full((L,), i + poff, jnp.int32)
            for t in range(C // L):
                sl = pl.ds(t * L, L)
                f00 = r00[i, sl]
                f01 = r01[i, sl]
                f10 = r10[i, sl]
                f11 = r11[i, sl]
                l0 = f00 + aw * (f01 - f00)
                l1 = f10 + aw * (f11 - f10)
                acc = l0 + ah * (l1 - l0)
                plsc.store_scatter(oc, [t * L + iota, col], acc)
            return carry

        lax.fori_loop(0, P, pt, 0, unroll=4)

    fire(0, sets[0])
    fire(1, sets[1])

    def quad(j, carry):
        g0 = 4 * j
        for q in range(4):
            g = g0 + q
            oc, osem = ocs[q // 2]
            tile_n0 = base + (g0 + (q // 2) * 2) * P

            if q % 2 == 0:
                @pl.when(j >= 1)
                def _(oc=oc, osem=osem, tile_n0=tile_n0):
                    pltpu.make_async_copy(
                        oc, out.at[b, :, pl.ds(tile_n0, 2 * P)], osem).wait()

            blend(g, sets[q % 2], oc, (q % 2) * P)

            @pl.when(g + 2 < CHUNKS)
            def _(g=g, q=q):
                fire(g + 2, sets[q % 2])

            if q % 2 == 1:
                pltpu.async_copy(
                    oc, out.at[b, :, pl.ds(tile_n0, 2 * P)], osem)
        return carry

    lax.fori_loop(0, QUADS, quad, 0)

    for k in range(2):
        oc, osem = ocs[k]
        tile_n0 = base + (CHUNKS - 4 + 2 * k) * P
        pltpu.make_async_copy(
            oc, out.at[b, :, pl.ds(tile_n0, 2 * P)], osem).wait()


def kernel(grid_in, pcds_ind):
    grid3 = grid_in.reshape(B, C, HW)
    table = _build_table(grid3)
    pc = pcds_ind.reshape(B, 2 * N)
    out = _sc_gather(table, pc)
    return out[..., None]
